# Initial kernel scaffold; baseline (speedup 1.0000x reference)
import jax
import jax.numpy as jnp
from jax.experimental import pallas as pl


def kernel(features, edge_indexs, W0, att_src0, att_dst0, b0, W1, att_src1, att_dst1, b1):
    raise NotImplementedError("write your pallas kernel here")


# speedup vs baseline: 23.9456x; 23.9456x over previous
import jax
import jax.numpy as jnp
from jax import lax
from jax.experimental import pallas as pl
from jax.experimental.pallas import tpu as pltpu
from jax.experimental.pallas import tpu_sc as plsc

N = 10000
D = 256
H = 8
C = 32
HALF = D // 2
HPC = H // 2
E_RAW = 160000
ET = E_RAW + N
NS = 16
B = 128
EPT = 10752
ET_PAD = NS * EPT
NB = EPT // B
NP = 10112
RPT = NP // NS
DW = NP * HPC
MBLK = 1000


def _sc_att_impl(as_lo, as_hi, ad_lo, ad_hi, src, dst, zden,
                 w_out, den_part,
                 as_v, ad_v, src_v, dst_v, wrow, den_l):
    c = lax.axis_index("c")
    s = lax.axis_index("s")
    i32 = jnp.int32
    iot = lax.iota(i32, 16)

    @pl.when(c == 0)
    def _():
        pltpu.sync_copy(as_lo, as_v)
        pltpu.sync_copy(ad_lo, ad_v)

    @pl.when(c == 1)
    def _():
        pltpu.sync_copy(as_hi, as_v)
        pltpu.sync_copy(ad_hi, ad_v)

    pltpu.sync_copy(zden, den_l)

    def batch_body(b, _):
        e0 = s * EPT + b * B
        pltpu.sync_copy(src.at[pl.ds(e0, B)], src_v)
        pltpu.sync_copy(dst.at[pl.ds(e0, B)], dst_v)

        for i in range(B // 16):
            sv = src_v[pl.ds(i * 16, 16)]
            dv = dst_v[pl.ds(i * 16, 16)]
            kvec = (jnp.full((16,), i * 16, i32) + iot) * HPC
            for j in range(HPC):
                jv = jnp.full((16,), j, i32)
                asj = plsc.load_gather(as_v, [sv * HPC + jv])
                adj = plsc.load_gather(ad_v, [dv * HPC + jv])
                e = asj + adj
                w = jnp.exp(jnp.maximum(e, 0.2 * e))
                plsc.store_scatter(wrow, [kvec + jv], w)
                plsc.addupdate_scatter(den_l, [dv * HPC + jv], w)

        pltpu.sync_copy(wrow, w_out.at[pl.ds((c * ET_PAD + e0) * HPC,
                                             B * HPC)])
        return 0

    lax.fori_loop(0, NB, batch_body, 0)
    pltpu.sync_copy(den_l, den_part.at[pl.ds((c * NS + s) * DW, DW)])


def _sc_att(as_lo, as_hi, ad_lo, ad_hi, src, dst):
    f32 = jnp.float32
    mesh = plsc.VectorSubcoreMesh(core_axis_name="c", subcore_axis_name="s")
    run = pl.kernel(
        _sc_att_impl,
        out_type=(
            jax.ShapeDtypeStruct((2 * ET_PAD * HPC,), f32),
            jax.ShapeDtypeStruct((2 * NS * DW,), f32),
        ),
        mesh=mesh,
        compiler_params=pltpu.CompilerParams(needs_layout_passes=False),
        scratch_types=[
            pltpu.VMEM((N * HPC,), f32),
            pltpu.VMEM((N * HPC,), f32),
            pltpu.VMEM((B,), jnp.int32),
            pltpu.VMEM((B,), jnp.int32),
            pltpu.VMEM((B * HPC,), f32),
            pltpu.VMEM((DW,), f32),
        ],
    )
    zden = jnp.zeros((DW,), f32)
    return run(as_lo, as_hi, ad_lo, ad_hi, src, dst, zden)


def _sc_agg_impl(h_lo, h_hi, w_in, src, dst, zacc, agg_out,
                 src_v, dst_v, hrows, wbuf, agg_sh):
    c = lax.axis_index("c")
    s = lax.axis_index("s")
    i32 = jnp.int32

    pltpu.sync_copy(zacc, agg_sh.at[pl.ds(s * RPT, RPT)])
    plsc.subcore_barrier()

    def batch_body(b, _):
        e0 = s * EPT + b * B
        pltpu.sync_copy(src.at[pl.ds(e0, B)], src_v)
        pltpu.sync_copy(dst.at[pl.ds(e0, B)], dst_v)
        pltpu.sync_copy(w_in.at[pl.ds((c * ET_PAD + e0) * HPC, B * HPC)],
                        wbuf)

        @pl.when(c == 0)
        def _():
            pltpu.sync_copy(h_lo.at[src_v], hrows)

        @pl.when(c == 1)
        def _():
            pltpu.sync_copy(h_hi.at[src_v], hrows)

        def scale_body(k, _):
            for j in range(HPC):
                wv = plsc.load_gather(
                    wbuf, [jnp.full((16,), k * HPC + j, i32)])
                for half in range(2):
                    col = (2 * j + half) * 16
                    hrows[k, pl.ds(col, 16)] = (
                        hrows[k, pl.ds(col, 16)] * wv)
            return 0

        lax.fori_loop(0, B, scale_body, 0)

        pltpu.sync_copy(hrows, agg_sh.at[dst_v], add=True)
        return 0

    lax.fori_loop(0, NB, batch_body, 0)
    plsc.subcore_barrier()

    pltpu.sync_copy(agg_sh.at[pl.ds(s * RPT, RPT)],
                    agg_out.at[pl.ds(c * NP + s * RPT, RPT)])


def _sc_agg(h_lo, h_hi, w_flat, src, dst):
    f32 = jnp.float32
    mesh = plsc.VectorSubcoreMesh(core_axis_name="c", subcore_axis_name="s")
    run = pl.kernel(
        _sc_agg_impl,
        out_type=jax.ShapeDtypeStruct((2 * NP, HALF), f32),
        mesh=mesh,
        compiler_params=pltpu.CompilerParams(needs_layout_passes=False),
        scratch_types=[
            pltpu.VMEM((B,), jnp.int32),
            pltpu.VMEM((B,), jnp.int32),
            pltpu.VMEM((B, HALF), f32),
            pltpu.VMEM((B * HPC,), f32),
            pltpu.VMEM_SHARED((NP, HALF), f32),
        ],
    )
    zacc = jnp.zeros((RPT, HALF), f32)
    return run(h_lo, h_hi, w_flat, src, dst, zacc)


def _tc_prep_body(x_ref, w_ref, asrc_ref, adst_ref,
                  hlo_ref, hhi_ref, as_ref, ad_ref):
    h = jnp.dot(x_ref[...], w_ref[...], preferred_element_type=jnp.float32)
    hlo_ref[...] = h[:, :HALF]
    hhi_ref[...] = h[:, HALF:]
    as_ref[...] = jnp.dot(h, asrc_ref[...], preferred_element_type=jnp.float32)
    ad_ref[...] = jnp.dot(h, adst_ref[...], preferred_element_type=jnp.float32)


def _tc_prep(x, w, asrc_bd, adst_bd):
    f32 = jnp.float32
    return pl.pallas_call(
        _tc_prep_body,
        grid=(N // MBLK,),
        in_specs=[
            pl.BlockSpec((MBLK, D), lambda i: (i, 0)),
            pl.BlockSpec((D, D), lambda i: (0, 0)),
            pl.BlockSpec((D, H), lambda i: (0, 0)),
            pl.BlockSpec((D, H), lambda i: (0, 0)),
        ],
        out_specs=[
            pl.BlockSpec((MBLK, HALF), lambda i: (i, 0)),
            pl.BlockSpec((MBLK, HALF), lambda i: (i, 0)),
            pl.BlockSpec((MBLK, H), lambda i: (i, 0)),
            pl.BlockSpec((MBLK, H), lambda i: (i, 0)),
        ],
        out_shape=[
            jax.ShapeDtypeStruct((N, HALF), f32),
            jax.ShapeDtypeStruct((N, HALF), f32),
            jax.ShapeDtypeStruct((N, H), f32),
            jax.ShapeDtypeStruct((N, H), f32),
        ],
    )(x, w, asrc_bd, adst_bd)


def _normalize(agg_ref, den_ref, exp4_ref, b_ref):
    den = jnp.sum(den_ref[...], axis=1)
    recip = 1.0 / (den + 1e-16)
    rex_lo = jnp.dot(recip[0], exp4_ref[...],
                     preferred_element_type=jnp.float32)
    rex_hi = jnp.dot(recip[1], exp4_ref[...],
                     preferred_element_type=jnp.float32)
    b = b_ref[...]
    x_lo = agg_ref[0] * rex_lo + b[:HALF][None, :]
    x_hi = agg_ref[1] * rex_hi + b[HALF:][None, :]
    return x_lo, x_hi


def _tc_mid_body(agg_ref, den_ref, exp4_ref, b_ref, w_ref, asrc_ref, adst_ref,
                 hlo_ref, hhi_ref, as_ref, ad_ref):
    x_lo, x_hi = _normalize(agg_ref, den_ref, exp4_ref, b_ref)
    w = w_ref[...]
    h = (jnp.dot(x_lo, w[:HALF, :], preferred_element_type=jnp.float32)
         + jnp.dot(x_hi, w[HALF:, :], preferred_element_type=jnp.float32))
    hlo_ref[...] = h[:, :HALF]
    hhi_ref[...] = h[:, HALF:]
    as_ref[...] = jnp.dot(h, asrc_ref[...], preferred_element_type=jnp.float32)
    ad_ref[...] = jnp.dot(h, adst_ref[...], preferred_element_type=jnp.float32)


def _tc_mid(agg, den, exp4, b, w, asrc_bd, adst_bd):
    f32 = jnp.float32
    return pl.pallas_call(
        _tc_mid_body,
        grid=(N // MBLK,),
        in_specs=[
            pl.BlockSpec((2, MBLK, HALF), lambda i: (0, i, 0)),
            pl.BlockSpec((2, NS, MBLK, HPC), lambda i: (0, 0, i, 0)),
            pl.BlockSpec((HPC, HALF), lambda i: (0, 0)),
            pl.BlockSpec((D,), lambda i: (0,)),
            pl.BlockSpec((D, D), lambda i: (0, 0)),
            pl.BlockSpec((D, H), lambda i: (0, 0)),
            pl.BlockSpec((D, H), lambda i: (0, 0)),
        ],
        out_specs=[
            pl.BlockSpec((MBLK, HALF), lambda i: (i, 0)),
            pl.BlockSpec((MBLK, HALF), lambda i: (i, 0)),
            pl.BlockSpec((MBLK, H), lambda i: (i, 0)),
            pl.BlockSpec((MBLK, H), lambda i: (i, 0)),
        ],
        out_shape=[
            jax.ShapeDtypeStruct((N, HALF), f32),
            jax.ShapeDtypeStruct((N, HALF), f32),
            jax.ShapeDtypeStruct((N, H), f32),
            jax.ShapeDtypeStruct((N, H), f32),
        ],
    )(agg, den, exp4, b, w, asrc_bd, adst_bd)


def _tc_final_body(agg_ref, den_ref, exp4_ref, b_ref, out_ref):
    x_lo, x_hi = _normalize(agg_ref, den_ref, exp4_ref, b_ref)
    x = jnp.concatenate([x_lo, x_hi], axis=1)
    out_ref[...] = jax.nn.gelu(x, approximate=True)


def _tc_final(agg, den, exp4, b):
    return pl.pallas_call(
        _tc_final_body,
        grid=(N // MBLK,),
        in_specs=[
            pl.BlockSpec((2, MBLK, HALF), lambda i: (0, i, 0)),
            pl.BlockSpec((2, NS, MBLK, HPC), lambda i: (0, 0, i, 0)),
            pl.BlockSpec((HPC, HALF), lambda i: (0, 0)),
            pl.BlockSpec((D,), lambda i: (0,)),
        ],
        out_specs=pl.BlockSpec((MBLK, D), lambda i: (i, 0)),
        out_shape=jax.ShapeDtypeStruct((N, D), jnp.float32),
    )(agg, den, exp4, b)


def _block_diag_att(att):
    rows = jnp.arange(D)
    cols = rows // C
    return jnp.zeros((D, H), jnp.float32).at[rows, cols].set(att.reshape(D))


def kernel(features, edge_indexs, W0, att_src0, att_dst0, b0,
           W1, att_src1, att_dst1, b1):
    f32 = jnp.float32
    i32 = jnp.int32

    loop = jnp.arange(N, dtype=edge_indexs.dtype)
    pad = ET_PAD - ET
    src = jnp.concatenate([edge_indexs[0], loop,
                           jnp.zeros((pad,), edge_indexs.dtype)]).astype(i32)
    dst = jnp.concatenate([edge_indexs[1], loop,
                           jnp.full((pad,), N, edge_indexs.dtype)]).astype(i32)

    exp4 = jnp.repeat(jnp.eye(HPC, dtype=f32), C, axis=1)

    asrc0_bd = _block_diag_att(att_src0)
    adst0_bd = _block_diag_att(att_dst0)
    asrc1_bd = _block_diag_att(att_src1)
    adst1_bd = _block_diag_att(att_dst1)

    def layer(h_lo, h_hi, as_full, ad_full):
        as_lo = as_full[:, :HPC].reshape(-1)
        as_hi = as_full[:, HPC:].reshape(-1)
        ad_lo = ad_full[:, :HPC].reshape(-1)
        ad_hi = ad_full[:, HPC:].reshape(-1)
        w_flat, den_flat = _sc_att(as_lo, as_hi, ad_lo, ad_hi, src, dst)
        agg_flat = _sc_agg(h_lo, h_hi, w_flat, src, dst)
        agg = agg_flat.reshape(2, NP, HALF)
        den = den_flat.reshape(2, NS, NP, HPC)
        return agg, den

    h_lo0, h_hi0, as0, ad0 = _tc_prep(features, W0, asrc0_bd, adst0_bd)
    agg0, den0 = layer(h_lo0, h_hi0, as0, ad0)
    h_lo1, h_hi1, as1, ad1 = _tc_mid(agg0, den0, exp4, b0, W1,
                                     asrc1_bd, adst1_bd)
    agg1, den1 = layer(h_lo1, h_hi1, as1, ad1)
    return _tc_final(agg1, den1, exp4, b1)


# speedup vs baseline: 29.4886x; 1.2315x over previous
import jax
import jax.numpy as jnp
from jax import lax
from jax.experimental import pallas as pl
from jax.experimental.pallas import tpu as pltpu
from jax.experimental.pallas import tpu_sc as plsc

N = 10000
D = 256
H = 8
C = 32
HALF = D // 2
HPC = H // 2
E_RAW = 160000
ET = E_RAW + N
NS = 16
B = 128
EPT = 10752
ET_PAD = NS * EPT
NB = EPT // B
NP = 10112
RPT = NP // NS
DW = NP * HPC
MBLK = 1000


def _sc_att_impl(as_lo, as_hi, ad_lo, ad_hi, src, dst, zden,
                 w_out, den_part,
                 as_v, ad_v, src_v, dst_v, wrow, den_l):
    c = lax.axis_index("c")
    s = lax.axis_index("s")
    i32 = jnp.int32
    iot = lax.iota(i32, 16)

    @pl.when(c == 0)
    def _():
        pltpu.sync_copy(as_lo, as_v)
        pltpu.sync_copy(ad_lo, ad_v)

    @pl.when(c == 1)
    def _():
        pltpu.sync_copy(as_hi, as_v)
        pltpu.sync_copy(ad_hi, ad_v)

    pltpu.sync_copy(zden, den_l)

    def batch_body(b, _):
        e0 = s * EPT + b * B
        pltpu.sync_copy(src.at[pl.ds(e0, B)], src_v)
        pltpu.sync_copy(dst.at[pl.ds(e0, B)], dst_v)

        for i in range(B // 16):
            sv = src_v[pl.ds(i * 16, 16)]
            dv = dst_v[pl.ds(i * 16, 16)]
            kvec = (jnp.full((16,), i * 16, i32) + iot) * HPC
            for j in range(HPC):
                jv = jnp.full((16,), j, i32)
                asj = plsc.load_gather(as_v, [sv * HPC + jv])
                adj = plsc.load_gather(ad_v, [dv * HPC + jv])
                e = asj + adj
                w = jnp.exp(jnp.maximum(e, 0.2 * e))
                plsc.store_scatter(wrow, [kvec + jv], w)
                plsc.addupdate_scatter(den_l, [dv * HPC + jv], w)

        pltpu.sync_copy(wrow, w_out.at[pl.ds((c * ET_PAD + e0) * HPC,
                                             B * HPC)])
        return 0

    lax.fori_loop(0, NB, batch_body, 0)
    pltpu.sync_copy(den_l, den_part.at[pl.ds((c * NS + s) * DW, DW)])


def _sc_att(as_lo, as_hi, ad_lo, ad_hi, src, dst):
    f32 = jnp.float32
    mesh = plsc.VectorSubcoreMesh(core_axis_name="c", subcore_axis_name="s")
    run = pl.kernel(
        _sc_att_impl,
        out_type=(
            jax.ShapeDtypeStruct((2 * ET_PAD * HPC,), f32),
            jax.ShapeDtypeStruct((2 * NS * DW,), f32),
        ),
        mesh=mesh,
        compiler_params=pltpu.CompilerParams(needs_layout_passes=False),
        scratch_types=[
            pltpu.VMEM((N * HPC,), f32),
            pltpu.VMEM((N * HPC,), f32),
            pltpu.VMEM((B,), jnp.int32),
            pltpu.VMEM((B,), jnp.int32),
            pltpu.VMEM((B * HPC,), f32),
            pltpu.VMEM((DW,), f32),
        ],
    )
    zden = jnp.zeros((DW,), f32)
    return run(as_lo, as_hi, ad_lo, ad_hi, src, dst, zden)


def _sc_agg_impl(h_lo, h_hi, w_in, src, dst, zacc, agg_out,
                 src_v, dst_v, hrows, wbuf, agg_sh):
    c = lax.axis_index("c")
    s = lax.axis_index("s")
    i32 = jnp.int32

    pltpu.sync_copy(zacc, agg_sh.at[pl.ds(s * RPT, RPT)])
    plsc.subcore_barrier()

    def batch_body(b, _):
        e0 = s * EPT + b * B
        pltpu.sync_copy(src.at[pl.ds(e0, B)], src_v)
        pltpu.sync_copy(dst.at[pl.ds(e0, B)], dst_v)
        pltpu.sync_copy(w_in.at[pl.ds((c * ET_PAD + e0) * HPC, B * HPC)],
                        wbuf)

        @pl.when(c == 0)
        def _():
            pltpu.sync_copy(h_lo.at[src_v], hrows)

        @pl.when(c == 1)
        def _():
            pltpu.sync_copy(h_hi.at[src_v], hrows)

        def scale_body(k4, _):
            wv16 = wbuf[pl.ds(k4 * 16, 16)]
            for e in range(4):
                k = k4 * 4 + e
                for j in range(HPC):
                    wv = wv16[jnp.full((16,), e * HPC + j, i32)]
                    for half in range(2):
                        col = (2 * j + half) * 16
                        hrows[k, pl.ds(col, 16)] = (
                            hrows[k, pl.ds(col, 16)] * wv)
            return 0

        lax.fori_loop(0, B // 4, scale_body, 0)

        pltpu.sync_copy(hrows, agg_sh.at[dst_v], add=True)
        return 0

    lax.fori_loop(0, NB, batch_body, 0)
    plsc.subcore_barrier()

    pltpu.sync_copy(agg_sh.at[pl.ds(s * RPT, RPT)],
                    agg_out.at[pl.ds(c * NP + s * RPT, RPT)])


def _sc_agg(h_lo, h_hi, w_flat, src, dst):
    f32 = jnp.float32
    mesh = plsc.VectorSubcoreMesh(core_axis_name="c", subcore_axis_name="s")
    run = pl.kernel(
        _sc_agg_impl,
        out_type=jax.ShapeDtypeStruct((2 * NP, HALF), f32),
        mesh=mesh,
        compiler_params=pltpu.CompilerParams(needs_layout_passes=False),
        scratch_types=[
            pltpu.VMEM((B,), jnp.int32),
            pltpu.VMEM((B,), jnp.int32),
            pltpu.VMEM((B, HALF), f32),
            pltpu.VMEM((B * HPC,), f32),
            pltpu.VMEM_SHARED((NP, HALF), f32),
        ],
    )
    zacc = jnp.zeros((RPT, HALF), f32)
    return run(h_lo, h_hi, w_flat, src, dst, zacc)


def _tc_prep_body(x_ref, w_ref, asrc_ref, adst_ref,
                  hlo_ref, hhi_ref, as_ref, ad_ref):
    h = jnp.dot(x_ref[...], w_ref[...], preferred_element_type=jnp.float32)
    hlo_ref[...] = h[:, :HALF]
    hhi_ref[...] = h[:, HALF:]
    as_ref[...] = jnp.dot(h, asrc_ref[...], preferred_element_type=jnp.float32)
    ad_ref[...] = jnp.dot(h, adst_ref[...], preferred_element_type=jnp.float32)


def _tc_prep(x, w, asrc_bd, adst_bd):
    f32 = jnp.float32
    return pl.pallas_call(
        _tc_prep_body,
        grid=(N // MBLK,),
        in_specs=[
            pl.BlockSpec((MBLK, D), lambda i: (i, 0)),
            pl.BlockSpec((D, D), lambda i: (0, 0)),
            pl.BlockSpec((D, H), lambda i: (0, 0)),
            pl.BlockSpec((D, H), lambda i: (0, 0)),
        ],
        out_specs=[
            pl.BlockSpec((MBLK, HALF), lambda i: (i, 0)),
            pl.BlockSpec((MBLK, HALF), lambda i: (i, 0)),
            pl.BlockSpec((MBLK, H), lambda i: (i, 0)),
            pl.BlockSpec((MBLK, H), lambda i: (i, 0)),
        ],
        out_shape=[
            jax.ShapeDtypeStruct((N, HALF), f32),
            jax.ShapeDtypeStruct((N, HALF), f32),
            jax.ShapeDtypeStruct((N, H), f32),
            jax.ShapeDtypeStruct((N, H), f32),
        ],
    )(x, w, asrc_bd, adst_bd)


def _normalize(agg_ref, den_ref, exp4_ref, b_ref):
    den = jnp.sum(den_ref[...], axis=1)
    recip = 1.0 / (den + 1e-16)
    rex_lo = jnp.dot(recip[0], exp4_ref[...],
                     preferred_element_type=jnp.float32)
    rex_hi = jnp.dot(recip[1], exp4_ref[...],
                     preferred_element_type=jnp.float32)
    b = b_ref[...]
    x_lo = agg_ref[0] * rex_lo + b[:HALF][None, :]
    x_hi = agg_ref[1] * rex_hi + b[HALF:][None, :]
    return x_lo, x_hi


def _tc_mid_body(agg_ref, den_ref, exp4_ref, b_ref, w_ref, asrc_ref, adst_ref,
                 hlo_ref, hhi_ref, as_ref, ad_ref):
    x_lo, x_hi = _normalize(agg_ref, den_ref, exp4_ref, b_ref)
    w = w_ref[...]
    h = (jnp.dot(x_lo, w[:HALF, :], preferred_element_type=jnp.float32)
         + jnp.dot(x_hi, w[HALF:, :], preferred_element_type=jnp.float32))
    hlo_ref[...] = h[:, :HALF]
    hhi_ref[...] = h[:, HALF:]
    as_ref[...] = jnp.dot(h, asrc_ref[...], preferred_element_type=jnp.float32)
    ad_ref[...] = jnp.dot(h, adst_ref[...], preferred_element_type=jnp.float32)


def _tc_mid(agg, den, exp4, b, w, asrc_bd, adst_bd):
    f32 = jnp.float32
    return pl.pallas_call(
        _tc_mid_body,
        grid=(N // MBLK,),
        in_specs=[
            pl.BlockSpec((2, MBLK, HALF), lambda i: (0, i, 0)),
            pl.BlockSpec((2, NS, MBLK, HPC), lambda i: (0, 0, i, 0)),
            pl.BlockSpec((HPC, HALF), lambda i: (0, 0)),
            pl.BlockSpec((D,), lambda i: (0,)),
            pl.BlockSpec((D, D), lambda i: (0, 0)),
            pl.BlockSpec((D, H), lambda i: (0, 0)),
            pl.BlockSpec((D, H), lambda i: (0, 0)),
        ],
        out_specs=[
            pl.BlockSpec((MBLK, HALF), lambda i: (i, 0)),
            pl.BlockSpec((MBLK, HALF), lambda i: (i, 0)),
            pl.BlockSpec((MBLK, H), lambda i: (i, 0)),
            pl.BlockSpec((MBLK, H), lambda i: (i, 0)),
        ],
        out_shape=[
            jax.ShapeDtypeStruct((N, HALF), f32),
            jax.ShapeDtypeStruct((N, HALF), f32),
            jax.ShapeDtypeStruct((N, H), f32),
            jax.ShapeDtypeStruct((N, H), f32),
        ],
    )(agg, den, exp4, b, w, asrc_bd, adst_bd)


def _tc_final_body(agg_ref, den_ref, exp4_ref, b_ref, out_ref):
    x_lo, x_hi = _normalize(agg_ref, den_ref, exp4_ref, b_ref)
    x = jnp.concatenate([x_lo, x_hi], axis=1)
    out_ref[...] = jax.nn.gelu(x, approximate=True)


def _tc_final(agg, den, exp4, b):
    return pl.pallas_call(
        _tc_final_body,
        grid=(N // MBLK,),
        in_specs=[
            pl.BlockSpec((2, MBLK, HALF), lambda i: (0, i, 0)),
            pl.BlockSpec((2, NS, MBLK, HPC), lambda i: (0, 0, i, 0)),
            pl.BlockSpec((HPC, HALF), lambda i: (0, 0)),
            pl.BlockSpec((D,), lambda i: (0,)),
        ],
        out_specs=pl.BlockSpec((MBLK, D), lambda i: (i, 0)),
        out_shape=jax.ShapeDtypeStruct((N, D), jnp.float32),
    )(agg, den, exp4, b)


def _block_diag_att(att):
    rows = jnp.arange(D)
    cols = rows // C
    return jnp.zeros((D, H), jnp.float32).at[rows, cols].set(att.reshape(D))


def kernel(features, edge_indexs, W0, att_src0, att_dst0, b0,
           W1, att_src1, att_dst1, b1):
    f32 = jnp.float32
    i32 = jnp.int32

    loop = jnp.arange(N, dtype=edge_indexs.dtype)
    pad = ET_PAD - ET
    src = jnp.concatenate([edge_indexs[0], loop,
                           jnp.zeros((pad,), edge_indexs.dtype)]).astype(i32)
    dst = jnp.concatenate([edge_indexs[1], loop,
                           jnp.full((pad,), N, edge_indexs.dtype)]).astype(i32)

    exp4 = jnp.repeat(jnp.eye(HPC, dtype=f32), C, axis=1)

    asrc0_bd = _block_diag_att(att_src0)
    adst0_bd = _block_diag_att(att_dst0)
    asrc1_bd = _block_diag_att(att_src1)
    adst1_bd = _block_diag_att(att_dst1)

    def layer(h_lo, h_hi, as_full, ad_full):
        as_lo = as_full[:, :HPC].reshape(-1)
        as_hi = as_full[:, HPC:].reshape(-1)
        ad_lo = ad_full[:, :HPC].reshape(-1)
        ad_hi = ad_full[:, HPC:].reshape(-1)
        w_flat, den_flat = _sc_att(as_lo, as_hi, ad_lo, ad_hi, src, dst)
        agg_flat = _sc_agg(h_lo, h_hi, w_flat, src, dst)
        agg = agg_flat.reshape(2, NP, HALF)
        den = den_flat.reshape(2, NS, NP, HPC)
        return agg, den

    h_lo0, h_hi0, as0, ad0 = _tc_prep(features, W0, asrc0_bd, adst0_bd)
    agg0, den0 = layer(h_lo0, h_hi0, as0, ad0)
    h_lo1, h_hi1, as1, ad1 = _tc_mid(agg0, den0, exp4, b0, W1,
                                     asrc1_bd, adst1_bd)
    agg1, den1 = layer(h_lo1, h_hi1, as1, ad1)
    return _tc_final(agg1, den1, exp4, b1)


# speedup vs baseline: 37.2464x; 1.2631x over previous
import jax
import jax.numpy as jnp
from jax import lax
from jax.experimental import pallas as pl
from jax.experimental.pallas import tpu as pltpu
from jax.experimental.pallas import tpu_sc as plsc

N = 10000
D = 256
H = 8
C = 32
HALF = D // 2
HPC = H // 2
E_RAW = 160000
ET = E_RAW + N
NS = 16
B = 128
EPT = 10752
ET_PAD = NS * EPT
NB = EPT // B
NP = 10112
RPT = NP // NS
DW = NP * HPC
MBLK = 1000


def _sc_att_impl(as_lo, as_hi, ad_lo, ad_hi, src, dst, zden,
                 w_out, den_part,
                 as_v, ad_v, src_v, dst_v, wrow, den_l):
    c = lax.axis_index("c")
    s = lax.axis_index("s")
    i32 = jnp.int32
    iot = lax.iota(i32, 16)

    @pl.when(c == 0)
    def _():
        pltpu.sync_copy(as_lo, as_v)
        pltpu.sync_copy(ad_lo, ad_v)

    @pl.when(c == 1)
    def _():
        pltpu.sync_copy(as_hi, as_v)
        pltpu.sync_copy(ad_hi, ad_v)

    pltpu.sync_copy(zden, den_l)

    def batch_body(b, _):
        e0 = s * EPT + b * B
        pltpu.sync_copy(src.at[pl.ds(e0, B)], src_v)
        pltpu.sync_copy(dst.at[pl.ds(e0, B)], dst_v)

        for i in range(B // 16):
            sv = src_v[pl.ds(i * 16, 16)]
            dv = dst_v[pl.ds(i * 16, 16)]
            kvec = (jnp.full((16,), i * 16, i32) + iot) * HPC
            for j in range(HPC):
                jv = jnp.full((16,), j, i32)
                asj = plsc.load_gather(as_v, [sv * HPC + jv])
                adj = plsc.load_gather(ad_v, [dv * HPC + jv])
                e = asj + adj
                w = jnp.exp(jnp.maximum(e, 0.2 * e))
                plsc.store_scatter(wrow, [kvec + jv], w)
                plsc.addupdate_scatter(den_l, [dv * HPC + jv], w)

        pltpu.sync_copy(wrow, w_out.at[pl.ds((c * ET_PAD + e0) * HPC,
                                             B * HPC)])
        return 0

    lax.fori_loop(0, NB, batch_body, 0)
    pltpu.sync_copy(den_l, den_part.at[pl.ds((c * NS + s) * DW, DW)])


def _sc_att(as_lo, as_hi, ad_lo, ad_hi, src, dst):
    f32 = jnp.float32
    mesh = plsc.VectorSubcoreMesh(core_axis_name="c", subcore_axis_name="s")
    run = pl.kernel(
        _sc_att_impl,
        out_type=(
            jax.ShapeDtypeStruct((2 * ET_PAD * HPC,), f32),
            jax.ShapeDtypeStruct((2 * NS * DW,), f32),
        ),
        mesh=mesh,
        compiler_params=pltpu.CompilerParams(needs_layout_passes=False),
        scratch_types=[
            pltpu.VMEM((N * HPC,), f32),
            pltpu.VMEM((N * HPC,), f32),
            pltpu.VMEM((B,), jnp.int32),
            pltpu.VMEM((B,), jnp.int32),
            pltpu.VMEM((B * HPC,), f32),
            pltpu.VMEM((DW,), f32),
        ],
    )
    zden = jnp.zeros((DW,), f32)
    return run(as_lo, as_hi, ad_lo, ad_hi, src, dst, zden)


def _sc_agg_impl(h_lo, h_hi, w_in, src, dst, zacc, agg_out,
                 src_va, src_vb, src_vc, dst_va, dst_vb, dst_vc,
                 wbuf_a, wbuf_b, wbuf_c, hrows0, hrows1, sem_g, sem_s,
                 agg_sh):
    c = lax.axis_index("c")
    s = lax.axis_index("s")
    i32 = jnp.int32
    hr = (hrows0, hrows1)
    srcv = (src_va, src_vb, src_vc)
    dstv = (dst_va, dst_vb, dst_vc)
    wbufs = (wbuf_a, wbuf_b, wbuf_c)

    pltpu.sync_copy(zacc, agg_sh.at[pl.ds(s * RPT, RPT)])
    plsc.subcore_barrier()

    def load_bufs(t, p3):
        e0 = s * EPT + t * B
        pltpu.sync_copy(src.at[pl.ds(e0, B)], srcv[p3])
        pltpu.sync_copy(dst.at[pl.ds(e0, B)], dstv[p3])
        pltpu.sync_copy(w_in.at[pl.ds((c * ET_PAD + e0) * HPC, B * HPC)],
                        wbufs[p3])

    def issue_gather(p3, p2):
        @pl.when(c == 0)
        def _():
            pltpu.async_copy(h_lo.at[srcv[p3]], hr[p2], sem_g)

        @pl.when(c == 1)
        def _():
            pltpu.async_copy(h_hi.at[srcv[p3]], hr[p2], sem_g)

    def wait_gather(p3, p2):
        pltpu.make_async_copy(h_lo.at[srcv[p3]], hr[p2], sem_g).wait()

    def issue_scatter(p3, p2):
        pltpu.async_copy(hr[p2], agg_sh.at[dstv[p3]], sem_s, add=True)

    def wait_scatter(p3, p2):
        pltpu.make_async_copy(hr[p2], agg_sh.at[dstv[p3]],
                              sem_s).wait()

    def scale(p3, p2):
        rows = hr[p2]

        def scale_body(k4, _):
            wv16 = wbufs[p3][pl.ds(k4 * 16, 16)]
            for e in range(4):
                k = k4 * 4 + e
                for j in range(HPC):
                    wv = wv16[jnp.full((16,), e * HPC + j, i32)]
                    for half in range(2):
                        col = (2 * j + half) * 16
                        rows[k, pl.ds(col, 16)] = (
                            rows[k, pl.ds(col, 16)] * wv)
            return 0

        lax.fori_loop(0, B // 4, scale_body, 0)

    def body(t, u, has_next=True, first=False):
        p2, p3 = u % 2, u % 3
        n2, n3 = (u + 1) % 2, (u + 1) % 3
        if has_next:
            load_bufs(t + 1, n3)
        if not first:
            wait_scatter((u - 1) % 3, n2)
        if has_next:
            issue_gather(n3, n2)
        wait_gather(p3, p2)
        scale(p3, p2)
        issue_scatter(p3, p2)

    load_bufs(0, 0)
    issue_gather(0, 0)
    body(jnp.int32(0), 0, first=True)

    def main_body(q, _):
        t0 = 1 + 6 * q
        for uu in range(6):
            body(t0 + uu, 1 + uu)
        return 0

    lax.fori_loop(0, 13, main_body, 0)

    for t in range(79, 83):
        body(jnp.int32(t), t)
    body(jnp.int32(83), 83, has_next=False)
    wait_scatter(83 % 3, 83 % 2)

    plsc.subcore_barrier()
    pltpu.sync_copy(agg_sh.at[pl.ds(s * RPT, RPT)],
                    agg_out.at[pl.ds(c * NP + s * RPT, RPT)])


def _sc_agg(h_lo, h_hi, w_flat, src, dst):
    f32 = jnp.float32
    mesh = plsc.VectorSubcoreMesh(core_axis_name="c", subcore_axis_name="s")
    run = pl.kernel(
        _sc_agg_impl,
        out_type=jax.ShapeDtypeStruct((2 * NP, HALF), f32),
        mesh=mesh,
        compiler_params=pltpu.CompilerParams(needs_layout_passes=False),
        scratch_types=[
            pltpu.VMEM((B,), jnp.int32),
            pltpu.VMEM((B,), jnp.int32),
            pltpu.VMEM((B,), jnp.int32),
            pltpu.VMEM((B,), jnp.int32),
            pltpu.VMEM((B,), jnp.int32),
            pltpu.VMEM((B,), jnp.int32),
            pltpu.VMEM((B * HPC,), f32),
            pltpu.VMEM((B * HPC,), f32),
            pltpu.VMEM((B * HPC,), f32),
            pltpu.VMEM((B, HALF), f32),
            pltpu.VMEM((B, HALF), f32),
            pltpu.SemaphoreType.DMA,
            pltpu.SemaphoreType.DMA,
            pltpu.VMEM_SHARED((NP, HALF), f32),
        ],
    )
    zacc = jnp.zeros((RPT, HALF), f32)
    return run(h_lo, h_hi, w_flat, src, dst, zacc)


def _tc_prep_body(x_ref, w_ref, asrc_ref, adst_ref,
                  hlo_ref, hhi_ref, as_ref, ad_ref):
    h = jnp.dot(x_ref[...], w_ref[...], preferred_element_type=jnp.float32)
    hlo_ref[...] = h[:, :HALF]
    hhi_ref[...] = h[:, HALF:]
    as_ref[...] = jnp.dot(h, asrc_ref[...], preferred_element_type=jnp.float32)
    ad_ref[...] = jnp.dot(h, adst_ref[...], preferred_element_type=jnp.float32)


def _tc_prep(x, w, asrc_bd, adst_bd):
    f32 = jnp.float32
    return pl.pallas_call(
        _tc_prep_body,
        grid=(N // MBLK,),
        in_specs=[
            pl.BlockSpec((MBLK, D), lambda i: (i, 0)),
            pl.BlockSpec((D, D), lambda i: (0, 0)),
            pl.BlockSpec((D, H), lambda i: (0, 0)),
            pl.BlockSpec((D, H), lambda i: (0, 0)),
        ],
        out_specs=[
            pl.BlockSpec((MBLK, HALF), lambda i: (i, 0)),
            pl.BlockSpec((MBLK, HALF), lambda i: (i, 0)),
            pl.BlockSpec((MBLK, H), lambda i: (i, 0)),
            pl.BlockSpec((MBLK, H), lambda i: (i, 0)),
        ],
        out_shape=[
            jax.ShapeDtypeStruct((N, HALF), f32),
            jax.ShapeDtypeStruct((N, HALF), f32),
            jax.ShapeDtypeStruct((N, H), f32),
            jax.ShapeDtypeStruct((N, H), f32),
        ],
    )(x, w, asrc_bd, adst_bd)


def _normalize(agg_ref, den_ref, exp4_ref, b_ref):
    den = jnp.sum(den_ref[...], axis=1)
    recip = 1.0 / (den + 1e-16)
    rex_lo = jnp.dot(recip[0], exp4_ref[...],
                     preferred_element_type=jnp.float32)
    rex_hi = jnp.dot(recip[1], exp4_ref[...],
                     preferred_element_type=jnp.float32)
    b = b_ref[...]
    x_lo = agg_ref[0] * rex_lo + b[:HALF][None, :]
    x_hi = agg_ref[1] * rex_hi + b[HALF:][None, :]
    return x_lo, x_hi


def _tc_mid_body(agg_ref, den_ref, exp4_ref, b_ref, w_ref, asrc_ref, adst_ref,
                 hlo_ref, hhi_ref, as_ref, ad_ref):
    x_lo, x_hi = _normalize(agg_ref, den_ref, exp4_ref, b_ref)
    w = w_ref[...]
    h = (jnp.dot(x_lo, w[:HALF, :], preferred_element_type=jnp.float32)
         + jnp.dot(x_hi, w[HALF:, :], preferred_element_type=jnp.float32))
    hlo_ref[...] = h[:, :HALF]
    hhi_ref[...] = h[:, HALF:]
    as_ref[...] = jnp.dot(h, asrc_ref[...], preferred_element_type=jnp.float32)
    ad_ref[...] = jnp.dot(h, adst_ref[...], preferred_element_type=jnp.float32)


def _tc_mid(agg, den, exp4, b, w, asrc_bd, adst_bd):
    f32 = jnp.float32
    return pl.pallas_call(
        _tc_mid_body,
        grid=(N // MBLK,),
        in_specs=[
            pl.BlockSpec((2, MBLK, HALF), lambda i: (0, i, 0)),
            pl.BlockSpec((2, NS, MBLK, HPC), lambda i: (0, 0, i, 0)),
            pl.BlockSpec((HPC, HALF), lambda i: (0, 0)),
            pl.BlockSpec((D,), lambda i: (0,)),
            pl.BlockSpec((D, D), lambda i: (0, 0)),
            pl.BlockSpec((D, H), lambda i: (0, 0)),
            pl.BlockSpec((D, H), lambda i: (0, 0)),
        ],
        out_specs=[
            pl.BlockSpec((MBLK, HALF), lambda i: (i, 0)),
            pl.BlockSpec((MBLK, HALF), lambda i: (i, 0)),
            pl.BlockSpec((MBLK, H), lambda i: (i, 0)),
            pl.BlockSpec((MBLK, H), lambda i: (i, 0)),
        ],
        out_shape=[
            jax.ShapeDtypeStruct((N, HALF), f32),
            jax.ShapeDtypeStruct((N, HALF), f32),
            jax.ShapeDtypeStruct((N, H), f32),
            jax.ShapeDtypeStruct((N, H), f32),
        ],
    )(agg, den, exp4, b, w, asrc_bd, adst_bd)


def _tc_final_body(agg_ref, den_ref, exp4_ref, b_ref, out_ref):
    x_lo, x_hi = _normalize(agg_ref, den_ref, exp4_ref, b_ref)
    x = jnp.concatenate([x_lo, x_hi], axis=1)
    out_ref[...] = jax.nn.gelu(x, approximate=True)


def _tc_final(agg, den, exp4, b):
    return pl.pallas_call(
        _tc_final_body,
        grid=(N // MBLK,),
        in_specs=[
            pl.BlockSpec((2, MBLK, HALF), lambda i: (0, i, 0)),
            pl.BlockSpec((2, NS, MBLK, HPC), lambda i: (0, 0, i, 0)),
            pl.BlockSpec((HPC, HALF), lambda i: (0, 0)),
            pl.BlockSpec((D,), lambda i: (0,)),
        ],
        out_specs=pl.BlockSpec((MBLK, D), lambda i: (i, 0)),
        out_shape=jax.ShapeDtypeStruct((N, D), jnp.float32),
    )(agg, den, exp4, b)


def _block_diag_att(att):
    rows = jnp.arange(D)
    cols = rows // C
    return jnp.zeros((D, H), jnp.float32).at[rows, cols].set(att.reshape(D))


def kernel(features, edge_indexs, W0, att_src0, att_dst0, b0,
           W1, att_src1, att_dst1, b1):
    f32 = jnp.float32
    i32 = jnp.int32

    loop = jnp.arange(N, dtype=edge_indexs.dtype)
    pad = ET_PAD - ET
    src = jnp.concatenate([edge_indexs[0], loop,
                           jnp.zeros((pad,), edge_indexs.dtype)]).astype(i32)
    dst = jnp.concatenate([edge_indexs[1], loop,
                           jnp.full((pad,), N, edge_indexs.dtype)]).astype(i32)

    exp4 = jnp.repeat(jnp.eye(HPC, dtype=f32), C, axis=1)

    asrc0_bd = _block_diag_att(att_src0)
    adst0_bd = _block_diag_att(att_dst0)
    asrc1_bd = _block_diag_att(att_src1)
    adst1_bd = _block_diag_att(att_dst1)

    def layer(h_lo, h_hi, as_full, ad_full):
        as_lo = as_full[:, :HPC].reshape(-1)
        as_hi = as_full[:, HPC:].reshape(-1)
        ad_lo = ad_full[:, :HPC].reshape(-1)
        ad_hi = ad_full[:, HPC:].reshape(-1)
        w_flat, den_flat = _sc_att(as_lo, as_hi, ad_lo, ad_hi, src, dst)
        agg_flat = _sc_agg(h_lo, h_hi, w_flat, src, dst)
        agg = agg_flat.reshape(2, NP, HALF)
        den = den_flat.reshape(2, NS, NP, HPC)
        return agg, den

    h_lo0, h_hi0, as0, ad0 = _tc_prep(features, W0, asrc0_bd, adst0_bd)
    agg0, den0 = layer(h_lo0, h_hi0, as0, ad0)
    h_lo1, h_hi1, as1, ad1 = _tc_mid(agg0, den0, exp4, b0, W1,
                                     asrc1_bd, adst1_bd)
    agg1, den1 = layer(h_lo1, h_hi1, as1, ad1)
    return _tc_final(agg1, den1, exp4, b1)


# speedup vs baseline: 43.0235x; 1.1551x over previous
import jax
import jax.numpy as jnp
from jax import lax
from jax.experimental import pallas as pl
from jax.experimental.pallas import tpu as pltpu
from jax.experimental.pallas import tpu_sc as plsc

N = 10000
D = 256
H = 8
C = 32
HALF = D // 2
HPC = H // 2
E_RAW = 160000
ET = E_RAW + N
NS = 16
B = 128
EPT = 10752
ET_PAD = NS * EPT
NB = EPT // B
NP = 10112
RPT = NP // NS
DW = NP * HPC
MBLK = 1000


def _sc_att_impl(as_lo, as_hi, ad_lo, ad_hi, src, dst, zden,
                 w_out, den_part,
                 as_v, ad_v, src_v0, src_v1, dst_v0, dst_v1,
                 wrow0, wrow1, sem_in, sem_out, den_l):
    c = lax.axis_index("c")
    s = lax.axis_index("s")
    i32 = jnp.int32
    iot = lax.iota(i32, 16)
    srcv = (src_v0, src_v1)
    dstv = (dst_v0, dst_v1)
    wr = (wrow0, wrow1)

    @pl.when(c == 0)
    def _():
        pltpu.sync_copy(as_lo, as_v)
        pltpu.sync_copy(ad_lo, ad_v)

    @pl.when(c == 1)
    def _():
        pltpu.sync_copy(as_hi, as_v)
        pltpu.sync_copy(ad_hi, ad_v)

    pltpu.sync_copy(zden, den_l)

    def issue_loads(t, p):
        e0 = s * EPT + t * B
        pltpu.async_copy(src.at[pl.ds(e0, B)], srcv[p], sem_in)
        pltpu.async_copy(dst.at[pl.ds(e0, B)], dstv[p], sem_in)

    def wait_loads(t, p):
        e0 = s * EPT + t * B
        pltpu.make_async_copy(src.at[pl.ds(e0, B)], srcv[p], sem_in).wait()
        pltpu.make_async_copy(dst.at[pl.ds(e0, B)], dstv[p], sem_in).wait()

    def out_desc(t, p):
        e0 = s * EPT + t * B
        return pltpu.make_async_copy(
            wr[p], w_out.at[pl.ds((c * ET_PAD + e0) * HPC, B * HPC)],
            sem_out)

    def compute(t, p):
        for i in range(B // 16):
            sv = srcv[p][pl.ds(i * 16, 16)]
            dv = dstv[p][pl.ds(i * 16, 16)]
            kvec = (jnp.full((16,), i * 16, i32) + iot) * HPC
            for j in range(HPC):
                jv = jnp.full((16,), j, i32)
                asj = plsc.load_gather(as_v, [sv * HPC + jv])
                adj = plsc.load_gather(ad_v, [dv * HPC + jv])
                e = asj + adj
                w = jnp.exp(jnp.maximum(e, 0.2 * e))
                plsc.store_scatter(wr[p], [kvec + jv], w)
                plsc.addupdate_scatter(den_l, [dv * HPC + jv], w)

    def body(t, u, has_next=True, first=False, wait_prev_out=True):
        p, np_ = u % 2, (u + 1) % 2
        if has_next:
            issue_loads(t + 1, np_)
        wait_loads(t, p)
        if wait_prev_out and not first:
            out_desc(t - 2, p).wait()
        compute(t, p)
        out_desc(t, p).start()

    issue_loads(jnp.int32(0), 0)
    body(jnp.int32(0), 0, first=True)
    body(jnp.int32(1), 1, first=True)

    def main_body(q, _):
        t0 = 2 + 2 * q
        body(t0, 0)
        body(t0 + 1, 1)
        return 0

    lax.fori_loop(0, (NB - 4) // 2, main_body, 0)

    body(jnp.int32(NB - 2), 0)
    body(jnp.int32(NB - 1), 1, has_next=False)
    out_desc(NB - 2, 0).wait()
    out_desc(NB - 1, 1).wait()

    pltpu.sync_copy(den_l, den_part.at[pl.ds((c * NS + s) * DW, DW)])


def _sc_att(as_lo, as_hi, ad_lo, ad_hi, src, dst):
    f32 = jnp.float32
    mesh = plsc.VectorSubcoreMesh(core_axis_name="c", subcore_axis_name="s")
    run = pl.kernel(
        _sc_att_impl,
        out_type=(
            jax.ShapeDtypeStruct((2 * ET_PAD * HPC,), f32),
            jax.ShapeDtypeStruct((2 * NS * DW,), f32),
        ),
        mesh=mesh,
        compiler_params=pltpu.CompilerParams(needs_layout_passes=False),
        scratch_types=[
            pltpu.VMEM((N * HPC,), f32),
            pltpu.VMEM((N * HPC,), f32),
            pltpu.VMEM((B,), jnp.int32),
            pltpu.VMEM((B,), jnp.int32),
            pltpu.VMEM((B,), jnp.int32),
            pltpu.VMEM((B,), jnp.int32),
            pltpu.VMEM((B * HPC,), f32),
            pltpu.VMEM((B * HPC,), f32),
            pltpu.SemaphoreType.DMA,
            pltpu.SemaphoreType.DMA,
            pltpu.VMEM((DW,), f32),
        ],
    )
    zden = jnp.zeros((DW,), f32)
    return run(as_lo, as_hi, ad_lo, ad_hi, src, dst, zden)


def _sc_agg_impl(h_lo, h_hi, w_in, src, dst, zacc, agg_out,
                 src_va, src_vb, src_vc, dst_va, dst_vb, dst_vc,
                 wbuf_a, wbuf_b, wbuf_c, hrows0, hrows1, sem_g, sem_s,
                 agg_sh):
    c = lax.axis_index("c")
    s = lax.axis_index("s")
    i32 = jnp.int32
    hr = (hrows0, hrows1)
    srcv = (src_va, src_vb, src_vc)
    dstv = (dst_va, dst_vb, dst_vc)
    wbufs = (wbuf_a, wbuf_b, wbuf_c)

    pltpu.sync_copy(zacc, agg_sh.at[pl.ds(s * RPT, RPT)])
    plsc.subcore_barrier()

    def load_bufs(t, p3):
        e0 = s * EPT + t * B
        pltpu.sync_copy(src.at[pl.ds(e0, B)], srcv[p3])
        pltpu.sync_copy(dst.at[pl.ds(e0, B)], dstv[p3])
        pltpu.sync_copy(w_in.at[pl.ds((c * ET_PAD + e0) * HPC, B * HPC)],
                        wbufs[p3])

    def issue_gather(p3, p2):
        @pl.when(c == 0)
        def _():
            pltpu.async_copy(h_lo.at[srcv[p3]], hr[p2], sem_g)

        @pl.when(c == 1)
        def _():
            pltpu.async_copy(h_hi.at[srcv[p3]], hr[p2], sem_g)

    def wait_gather(p3, p2):
        pltpu.make_async_copy(h_lo.at[srcv[p3]], hr[p2], sem_g).wait()

    def issue_scatter(p3, p2):
        pltpu.async_copy(hr[p2], agg_sh.at[dstv[p3]], sem_s, add=True)

    def wait_scatter(p3, p2):
        pltpu.make_async_copy(hr[p2], agg_sh.at[dstv[p3]],
                              sem_s).wait()

    def scale(p3, p2):
        rows = hr[p2]

        def scale_body(k4, _):
            wv16 = wbufs[p3][pl.ds(k4 * 16, 16)]
            for e in range(4):
                k = k4 * 4 + e
                for j in range(HPC):
                    wv = wv16[jnp.full((16,), e * HPC + j, i32)]
                    for half in range(2):
                        col = (2 * j + half) * 16
                        rows[k, pl.ds(col, 16)] = (
                            rows[k, pl.ds(col, 16)] * wv)
            return 0

        lax.fori_loop(0, B // 4, scale_body, 0)

    def body(t, u, has_next=True, first=False):
        p2, p3 = u % 2, u % 3
        n2, n3 = (u + 1) % 2, (u + 1) % 3
        if has_next:
            load_bufs(t + 1, n3)
        if not first:
            wait_scatter((u - 1) % 3, n2)
        if has_next:
            issue_gather(n3, n2)
        wait_gather(p3, p2)
        scale(p3, p2)
        issue_scatter(p3, p2)

    load_bufs(0, 0)
    issue_gather(0, 0)
    body(jnp.int32(0), 0, first=True)

    def main_body(q, _):
        t0 = 1 + 6 * q
        for uu in range(6):
            body(t0 + uu, 1 + uu)
        return 0

    lax.fori_loop(0, 13, main_body, 0)

    for t in range(79, 83):
        body(jnp.int32(t), t)
    body(jnp.int32(83), 83, has_next=False)
    wait_scatter(83 % 3, 83 % 2)

    plsc.subcore_barrier()
    pltpu.sync_copy(agg_sh.at[pl.ds(s * RPT, RPT)],
                    agg_out.at[pl.ds(c * NP + s * RPT, RPT)])


def _sc_agg(h_lo, h_hi, w_flat, src, dst):
    f32 = jnp.float32
    mesh = plsc.VectorSubcoreMesh(core_axis_name="c", subcore_axis_name="s")
    run = pl.kernel(
        _sc_agg_impl,
        out_type=jax.ShapeDtypeStruct((2 * NP, HALF), f32),
        mesh=mesh,
        compiler_params=pltpu.CompilerParams(needs_layout_passes=False),
        scratch_types=[
            pltpu.VMEM((B,), jnp.int32),
            pltpu.VMEM((B,), jnp.int32),
            pltpu.VMEM((B,), jnp.int32),
            pltpu.VMEM((B,), jnp.int32),
            pltpu.VMEM((B,), jnp.int32),
            pltpu.VMEM((B,), jnp.int32),
            pltpu.VMEM((B * HPC,), f32),
            pltpu.VMEM((B * HPC,), f32),
            pltpu.VMEM((B * HPC,), f32),
            pltpu.VMEM((B, HALF), f32),
            pltpu.VMEM((B, HALF), f32),
            pltpu.SemaphoreType.DMA,
            pltpu.SemaphoreType.DMA,
            pltpu.VMEM_SHARED((NP, HALF), f32),
        ],
    )
    zacc = jnp.zeros((RPT, HALF), f32)
    return run(h_lo, h_hi, w_flat, src, dst, zacc)


def _tc_prep_body(x_ref, w_ref, asrc_ref, adst_ref,
                  hlo_ref, hhi_ref, as_ref, ad_ref):
    h = jnp.dot(x_ref[...], w_ref[...], preferred_element_type=jnp.float32)
    hlo_ref[...] = h[:, :HALF]
    hhi_ref[...] = h[:, HALF:]
    as_ref[...] = jnp.dot(h, asrc_ref[...], preferred_element_type=jnp.float32)
    ad_ref[...] = jnp.dot(h, adst_ref[...], preferred_element_type=jnp.float32)


def _tc_prep(x, w, asrc_bd, adst_bd):
    f32 = jnp.float32
    return pl.pallas_call(
        _tc_prep_body,
        grid=(N // MBLK,),
        in_specs=[
            pl.BlockSpec((MBLK, D), lambda i: (i, 0)),
            pl.BlockSpec((D, D), lambda i: (0, 0)),
            pl.BlockSpec((D, H), lambda i: (0, 0)),
            pl.BlockSpec((D, H), lambda i: (0, 0)),
        ],
        out_specs=[
            pl.BlockSpec((MBLK, HALF), lambda i: (i, 0)),
            pl.BlockSpec((MBLK, HALF), lambda i: (i, 0)),
            pl.BlockSpec((MBLK, H), lambda i: (i, 0)),
            pl.BlockSpec((MBLK, H), lambda i: (i, 0)),
        ],
        out_shape=[
            jax.ShapeDtypeStruct((N, HALF), f32),
            jax.ShapeDtypeStruct((N, HALF), f32),
            jax.ShapeDtypeStruct((N, H), f32),
            jax.ShapeDtypeStruct((N, H), f32),
        ],
    )(x, w, asrc_bd, adst_bd)


def _normalize(agg_ref, den_ref, exp4_ref, b_ref):
    den = jnp.sum(den_ref[...], axis=1)
    recip = 1.0 / (den + 1e-16)
    rex_lo = jnp.dot(recip[0], exp4_ref[...],
                     preferred_element_type=jnp.float32)
    rex_hi = jnp.dot(recip[1], exp4_ref[...],
                     preferred_element_type=jnp.float32)
    b = b_ref[...]
    x_lo = agg_ref[0] * rex_lo + b[:HALF][None, :]
    x_hi = agg_ref[1] * rex_hi + b[HALF:][None, :]
    return x_lo, x_hi


def _tc_mid_body(agg_ref, den_ref, exp4_ref, b_ref, w_ref, asrc_ref, adst_ref,
                 hlo_ref, hhi_ref, as_ref, ad_ref):
    x_lo, x_hi = _normalize(agg_ref, den_ref, exp4_ref, b_ref)
    w = w_ref[...]
    h = (jnp.dot(x_lo, w[:HALF, :], preferred_element_type=jnp.float32)
         + jnp.dot(x_hi, w[HALF:, :], preferred_element_type=jnp.float32))
    hlo_ref[...] = h[:, :HALF]
    hhi_ref[...] = h[:, HALF:]
    as_ref[...] = jnp.dot(h, asrc_ref[...], preferred_element_type=jnp.float32)
    ad_ref[...] = jnp.dot(h, adst_ref[...], preferred_element_type=jnp.float32)


def _tc_mid(agg, den, exp4, b, w, asrc_bd, adst_bd):
    f32 = jnp.float32
    return pl.pallas_call(
        _tc_mid_body,
        grid=(N // MBLK,),
        in_specs=[
            pl.BlockSpec((2, MBLK, HALF), lambda i: (0, i, 0)),
            pl.BlockSpec((2, NS, MBLK, HPC), lambda i: (0, 0, i, 0)),
            pl.BlockSpec((HPC, HALF), lambda i: (0, 0)),
            pl.BlockSpec((D,), lambda i: (0,)),
            pl.BlockSpec((D, D), lambda i: (0, 0)),
            pl.BlockSpec((D, H), lambda i: (0, 0)),
            pl.BlockSpec((D, H), lambda i: (0, 0)),
        ],
        out_specs=[
            pl.BlockSpec((MBLK, HALF), lambda i: (i, 0)),
            pl.BlockSpec((MBLK, HALF), lambda i: (i, 0)),
            pl.BlockSpec((MBLK, H), lambda i: (i, 0)),
            pl.BlockSpec((MBLK, H), lambda i: (i, 0)),
        ],
        out_shape=[
            jax.ShapeDtypeStruct((N, HALF), f32),
            jax.ShapeDtypeStruct((N, HALF), f32),
            jax.ShapeDtypeStruct((N, H), f32),
            jax.ShapeDtypeStruct((N, H), f32),
        ],
    )(agg, den, exp4, b, w, asrc_bd, adst_bd)


def _tc_final_body(agg_ref, den_ref, exp4_ref, b_ref, out_ref):
    x_lo, x_hi = _normalize(agg_ref, den_ref, exp4_ref, b_ref)
    x = jnp.concatenate([x_lo, x_hi], axis=1)
    out_ref[...] = jax.nn.gelu(x, approximate=True)


def _tc_final(agg, den, exp4, b):
    return pl.pallas_call(
        _tc_final_body,
        grid=(N // MBLK,),
        in_specs=[
            pl.BlockSpec((2, MBLK, HALF), lambda i: (0, i, 0)),
            pl.BlockSpec((2, NS, MBLK, HPC), lambda i: (0, 0, i, 0)),
            pl.BlockSpec((HPC, HALF), lambda i: (0, 0)),
            pl.BlockSpec((D,), lambda i: (0,)),
        ],
        out_specs=pl.BlockSpec((MBLK, D), lambda i: (i, 0)),
        out_shape=jax.ShapeDtypeStruct((N, D), jnp.float32),
    )(agg, den, exp4, b)


def _block_diag_att(att):
    rows = jnp.arange(D)
    cols = rows // C
    return jnp.zeros((D, H), jnp.float32).at[rows, cols].set(att.reshape(D))


def kernel(features, edge_indexs, W0, att_src0, att_dst0, b0,
           W1, att_src1, att_dst1, b1):
    f32 = jnp.float32
    i32 = jnp.int32

    loop = jnp.arange(N, dtype=edge_indexs.dtype)
    pad = ET_PAD - ET
    src = jnp.concatenate([edge_indexs[0], loop,
                           jnp.zeros((pad,), edge_indexs.dtype)]).astype(i32)
    dst = jnp.concatenate([edge_indexs[1], loop,
                           jnp.full((pad,), N, edge_indexs.dtype)]).astype(i32)

    exp4 = jnp.repeat(jnp.eye(HPC, dtype=f32), C, axis=1)

    asrc0_bd = _block_diag_att(att_src0)
    adst0_bd = _block_diag_att(att_dst0)
    asrc1_bd = _block_diag_att(att_src1)
    adst1_bd = _block_diag_att(att_dst1)

    def layer(h_lo, h_hi, as_full, ad_full):
        as_lo = as_full[:, :HPC].reshape(-1)
        as_hi = as_full[:, HPC:].reshape(-1)
        ad_lo = ad_full[:, :HPC].reshape(-1)
        ad_hi = ad_full[:, HPC:].reshape(-1)
        w_flat, den_flat = _sc_att(as_lo, as_hi, ad_lo, ad_hi, src, dst)
        agg_flat = _sc_agg(h_lo, h_hi, w_flat, src, dst)
        agg = agg_flat.reshape(2, NP, HALF)
        den = den_flat.reshape(2, NS, NP, HPC)
        return agg, den

    h_lo0, h_hi0, as0, ad0 = _tc_prep(features, W0, asrc0_bd, adst0_bd)
    agg0, den0 = layer(h_lo0, h_hi0, as0, ad0)
    h_lo1, h_hi1, as1, ad1 = _tc_mid(agg0, den0, exp4, b0, W1,
                                     asrc1_bd, adst1_bd)
    agg1, den1 = layer(h_lo1, h_hi1, as1, ad1)
    return _tc_final(agg1, den1, exp4, b1)


# speedup vs baseline: 47.0046x; 1.0925x over previous
import jax
import jax.numpy as jnp
from jax import lax
from jax.experimental import pallas as pl
from jax.experimental.pallas import tpu as pltpu
from jax.experimental.pallas import tpu_sc as plsc

N = 10000
D = 256
H = 8
C = 32
HALF = D // 2
HPC = H // 2
E_RAW = 160000
ET = E_RAW + N
NS = 16
B = 128
EPT = 10752
ET_PAD = NS * EPT
NB = EPT // B
NP = 10112
RPT = NP // NS
DW = NP * HPC
MBLK = 1000


def _sc_att_impl(as_lo, as_hi, ad_lo, ad_hi, src, dst, zden,
                 w_out, den_part,
                 as_v, ad_v, src_v0, src_v1, dst_v0, dst_v1,
                 wrow0, wrow1, sem_in, sem_out, den_l):
    c = lax.axis_index("c")
    s = lax.axis_index("s")
    i32 = jnp.int32
    iot = lax.iota(i32, 16)
    srcv = (src_v0, src_v1)
    dstv = (dst_v0, dst_v1)
    wr = (wrow0, wrow1)

    @pl.when(c == 0)
    def _():
        pltpu.sync_copy(as_lo, as_v)
        pltpu.sync_copy(ad_lo, ad_v)

    @pl.when(c == 1)
    def _():
        pltpu.sync_copy(as_hi, as_v)
        pltpu.sync_copy(ad_hi, ad_v)

    pltpu.sync_copy(zden, den_l)

    def issue_loads(t, p):
        e0 = s * EPT + t * B
        pltpu.async_copy(src.at[pl.ds(e0, B)], srcv[p], sem_in)
        pltpu.async_copy(dst.at[pl.ds(e0, B)], dstv[p], sem_in)

    def wait_loads(t, p):
        e0 = s * EPT + t * B
        pltpu.make_async_copy(src.at[pl.ds(e0, B)], srcv[p], sem_in).wait()
        pltpu.make_async_copy(dst.at[pl.ds(e0, B)], dstv[p], sem_in).wait()

    def out_desc(t, p):
        e0 = s * EPT + t * B
        return pltpu.make_async_copy(
            wr[p], w_out.at[pl.ds((c * ET_PAD + e0) * HPC, B * HPC)],
            sem_out)

    def compute(t, p):
        for i in range(B // 16):
            sv = srcv[p][pl.ds(i * 16, 16)]
            dv = dstv[p][pl.ds(i * 16, 16)]
            kvec = (jnp.full((16,), i * 16, i32) + iot) * HPC
            for j in range(HPC):
                jv = jnp.full((16,), j, i32)
                asj = plsc.load_gather(as_v, [sv * HPC + jv])
                adj = plsc.load_gather(ad_v, [dv * HPC + jv])
                e = asj + adj
                w = jnp.exp(jnp.maximum(e, 0.2 * e))
                plsc.store_scatter(wr[p], [kvec + jv], w)
                plsc.addupdate_scatter(den_l, [dv * HPC + jv], w)

    def body(t, u, has_next=True, first=False, wait_prev_out=True):
        p, np_ = u % 2, (u + 1) % 2
        if has_next:
            issue_loads(t + 1, np_)
        wait_loads(t, p)
        if wait_prev_out and not first:
            out_desc(t - 2, p).wait()
        compute(t, p)
        out_desc(t, p).start()

    issue_loads(jnp.int32(0), 0)
    body(jnp.int32(0), 0, first=True)
    body(jnp.int32(1), 1, first=True)

    def main_body(q, _):
        t0 = 2 + 2 * q
        body(t0, 0)
        body(t0 + 1, 1)
        return 0

    lax.fori_loop(0, (NB - 4) // 2, main_body, 0)

    body(jnp.int32(NB - 2), 0)
    body(jnp.int32(NB - 1), 1, has_next=False)
    out_desc(NB - 2, 0).wait()
    out_desc(NB - 1, 1).wait()

    pltpu.sync_copy(den_l, den_part.at[pl.ds((c * NS + s) * DW, DW)])


def _sc_att(as_lo, as_hi, ad_lo, ad_hi, src, dst):
    f32 = jnp.float32
    mesh = plsc.VectorSubcoreMesh(core_axis_name="c", subcore_axis_name="s")
    run = pl.kernel(
        _sc_att_impl,
        out_type=(
            jax.ShapeDtypeStruct((2 * ET_PAD * HPC,), f32),
            jax.ShapeDtypeStruct((2 * NS * DW,), f32),
        ),
        mesh=mesh,
        compiler_params=pltpu.CompilerParams(needs_layout_passes=False),
        scratch_types=[
            pltpu.VMEM((N * HPC,), f32),
            pltpu.VMEM((N * HPC,), f32),
            pltpu.VMEM((B,), jnp.int32),
            pltpu.VMEM((B,), jnp.int32),
            pltpu.VMEM((B,), jnp.int32),
            pltpu.VMEM((B,), jnp.int32),
            pltpu.VMEM((B * HPC,), f32),
            pltpu.VMEM((B * HPC,), f32),
            pltpu.SemaphoreType.DMA,
            pltpu.SemaphoreType.DMA,
            pltpu.VMEM((DW,), f32),
        ],
    )
    zden = jnp.zeros((DW,), f32)
    return run(as_lo, as_hi, ad_lo, ad_hi, src, dst, zden)


def _sc_agg_impl(h_lo, h_hi, w_in, src, dst, zacc, agg_out,
                 src_va, src_vb, src_vc, dst_va, dst_vb, dst_vc,
                 wbuf_a, wbuf_b, wbuf_c, hrows0, hrows1, sem_g, sem_s,
                 sem_l, agg_sh):
    c = lax.axis_index("c")
    s = lax.axis_index("s")
    i32 = jnp.int32
    hr = (hrows0, hrows1)
    srcv = (src_va, src_vb, src_vc)
    dstv = (dst_va, dst_vb, dst_vc)
    wbufs = (wbuf_a, wbuf_b, wbuf_c)

    pltpu.sync_copy(zacc, agg_sh.at[pl.ds(s * RPT, RPT)])
    plsc.subcore_barrier()

    def load_descs(t, p3):
        e0 = s * EPT + t * B
        return (
            pltpu.make_async_copy(src.at[pl.ds(e0, B)], srcv[p3], sem_l),
            pltpu.make_async_copy(dst.at[pl.ds(e0, B)], dstv[p3], sem_l),
            pltpu.make_async_copy(
                w_in.at[pl.ds((c * ET_PAD + e0) * HPC, B * HPC)],
                wbufs[p3], sem_l),
        )

    def issue_loads(t, p3):
        for d in load_descs(t, p3):
            d.start()

    def wait_loads(t, p3):
        for d in load_descs(t, p3):
            d.wait()

    def issue_gather(p3, p2):
        @pl.when(c == 0)
        def _():
            pltpu.async_copy(h_lo.at[srcv[p3]], hr[p2], sem_g)

        @pl.when(c == 1)
        def _():
            pltpu.async_copy(h_hi.at[srcv[p3]], hr[p2], sem_g)

    def wait_gather(p3, p2):
        pltpu.make_async_copy(h_lo.at[srcv[p3]], hr[p2], sem_g).wait()

    def issue_scatter(p3, p2):
        pltpu.async_copy(hr[p2], agg_sh.at[dstv[p3]], sem_s, add=True)

    def wait_scatter(p3, p2):
        pltpu.make_async_copy(hr[p2], agg_sh.at[dstv[p3]],
                              sem_s).wait()

    def scale(p3, p2):
        rows = hr[p2]

        def scale_body(k4, _):
            wv16 = wbufs[p3][pl.ds(k4 * 16, 16)]
            for e in range(4):
                k = k4 * 4 + e
                for j in range(HPC):
                    wv = wv16[jnp.full((16,), e * HPC + j, i32)]
                    for half in range(2):
                        col = (2 * j + half) * 16
                        rows[k, pl.ds(col, 16)] = (
                            rows[k, pl.ds(col, 16)] * wv)
            return 0

        lax.fori_loop(0, B // 4, scale_body, 0)

    def body(t, u, has_next=True, first=False, prefetch=True):
        p2, p3 = u % 2, u % 3
        n2, n3 = (u + 1) % 2, (u + 1) % 3
        f3 = (u + 2) % 3
        if prefetch:
            issue_loads(t + 2, f3)
        if not first:
            wait_scatter((u - 1) % 3, n2)
        if has_next:
            wait_loads(t + 1, n3)
            issue_gather(n3, n2)
        wait_gather(p3, p2)
        scale(p3, p2)
        issue_scatter(p3, p2)

    issue_loads(jnp.int32(0), 0)
    issue_loads(jnp.int32(1), 1)
    wait_loads(jnp.int32(0), 0)
    issue_gather(0, 0)
    body(jnp.int32(0), 0, first=True)

    def main_body(q, _):
        t0 = 1 + 6 * q
        for uu in range(6):
            body(t0 + uu, 1 + uu)
        return 0

    lax.fori_loop(0, 13, main_body, 0)

    for t in range(79, 82):
        body(jnp.int32(t), t)
    body(jnp.int32(82), 82, prefetch=False)
    body(jnp.int32(83), 83, has_next=False, prefetch=False)
    wait_scatter(83 % 3, 83 % 2)

    plsc.subcore_barrier()
    pltpu.sync_copy(agg_sh.at[pl.ds(s * RPT, RPT)],
                    agg_out.at[pl.ds(c * NP + s * RPT, RPT)])


def _sc_agg(h_lo, h_hi, w_flat, src, dst):
    f32 = jnp.float32
    mesh = plsc.VectorSubcoreMesh(core_axis_name="c", subcore_axis_name="s")
    run = pl.kernel(
        _sc_agg_impl,
        out_type=jax.ShapeDtypeStruct((2 * NP, HALF), f32),
        mesh=mesh,
        compiler_params=pltpu.CompilerParams(needs_layout_passes=False),
        scratch_types=[
            pltpu.VMEM((B,), jnp.int32),
            pltpu.VMEM((B,), jnp.int32),
            pltpu.VMEM((B,), jnp.int32),
            pltpu.VMEM((B,), jnp.int32),
            pltpu.VMEM((B,), jnp.int32),
            pltpu.VMEM((B,), jnp.int32),
            pltpu.VMEM((B * HPC,), f32),
            pltpu.VMEM((B * HPC,), f32),
            pltpu.VMEM((B * HPC,), f32),
            pltpu.VMEM((B, HALF), f32),
            pltpu.VMEM((B, HALF), f32),
            pltpu.SemaphoreType.DMA,
            pltpu.SemaphoreType.DMA,
            pltpu.SemaphoreType.DMA,
            pltpu.VMEM_SHARED((NP, HALF), f32),
        ],
    )
    zacc = jnp.zeros((RPT, HALF), f32)
    return run(h_lo, h_hi, w_flat, src, dst, zacc)


def _tc_prep_body(x_ref, w_ref, asrc_ref, adst_ref,
                  hlo_ref, hhi_ref, as_ref, ad_ref):
    h = jnp.dot(x_ref[...], w_ref[...], preferred_element_type=jnp.float32)
    hlo_ref[...] = h[:, :HALF]
    hhi_ref[...] = h[:, HALF:]
    as_ref[...] = jnp.dot(h, asrc_ref[...], preferred_element_type=jnp.float32)
    ad_ref[...] = jnp.dot(h, adst_ref[...], preferred_element_type=jnp.float32)


def _tc_prep(x, w, asrc_bd, adst_bd):
    f32 = jnp.float32
    return pl.pallas_call(
        _tc_prep_body,
        grid=(N // MBLK,),
        in_specs=[
            pl.BlockSpec((MBLK, D), lambda i: (i, 0)),
            pl.BlockSpec((D, D), lambda i: (0, 0)),
            pl.BlockSpec((D, H), lambda i: (0, 0)),
            pl.BlockSpec((D, H), lambda i: (0, 0)),
        ],
        out_specs=[
            pl.BlockSpec((MBLK, HALF), lambda i: (i, 0)),
            pl.BlockSpec((MBLK, HALF), lambda i: (i, 0)),
            pl.BlockSpec((MBLK, H), lambda i: (i, 0)),
            pl.BlockSpec((MBLK, H), lambda i: (i, 0)),
        ],
        out_shape=[
            jax.ShapeDtypeStruct((N, HALF), f32),
            jax.ShapeDtypeStruct((N, HALF), f32),
            jax.ShapeDtypeStruct((N, H), f32),
            jax.ShapeDtypeStruct((N, H), f32),
        ],
    )(x, w, asrc_bd, adst_bd)


def _normalize(agg_ref, den_ref, exp4_ref, b_ref):
    den = jnp.sum(den_ref[...], axis=1)
    recip = 1.0 / (den + 1e-16)
    rex_lo = jnp.dot(recip[0], exp4_ref[...],
                     preferred_element_type=jnp.float32)
    rex_hi = jnp.dot(recip[1], exp4_ref[...],
                     preferred_element_type=jnp.float32)
    b = b_ref[...]
    x_lo = agg_ref[0] * rex_lo + b[:HALF][None, :]
    x_hi = agg_ref[1] * rex_hi + b[HALF:][None, :]
    return x_lo, x_hi


def _tc_mid_body(agg_ref, den_ref, exp4_ref, b_ref, w_ref, asrc_ref, adst_ref,
                 hlo_ref, hhi_ref, as_ref, ad_ref):
    x_lo, x_hi = _normalize(agg_ref, den_ref, exp4_ref, b_ref)
    w = w_ref[...]
    h = (jnp.dot(x_lo, w[:HALF, :], preferred_element_type=jnp.float32)
         + jnp.dot(x_hi, w[HALF:, :], preferred_element_type=jnp.float32))
    hlo_ref[...] = h[:, :HALF]
    hhi_ref[...] = h[:, HALF:]
    as_ref[...] = jnp.dot(h, asrc_ref[...], preferred_element_type=jnp.float32)
    ad_ref[...] = jnp.dot(h, adst_ref[...], preferred_element_type=jnp.float32)


def _tc_mid(agg, den, exp4, b, w, asrc_bd, adst_bd):
    f32 = jnp.float32
    return pl.pallas_call(
        _tc_mid_body,
        grid=(N // MBLK,),
        in_specs=[
            pl.BlockSpec((2, MBLK, HALF), lambda i: (0, i, 0)),
            pl.BlockSpec((2, NS, MBLK, HPC), lambda i: (0, 0, i, 0)),
            pl.BlockSpec((HPC, HALF), lambda i: (0, 0)),
            pl.BlockSpec((D,), lambda i: (0,)),
            pl.BlockSpec((D, D), lambda i: (0, 0)),
            pl.BlockSpec((D, H), lambda i: (0, 0)),
            pl.BlockSpec((D, H), lambda i: (0, 0)),
        ],
        out_specs=[
            pl.BlockSpec((MBLK, HALF), lambda i: (i, 0)),
            pl.BlockSpec((MBLK, HALF), lambda i: (i, 0)),
            pl.BlockSpec((MBLK, H), lambda i: (i, 0)),
            pl.BlockSpec((MBLK, H), lambda i: (i, 0)),
        ],
        out_shape=[
            jax.ShapeDtypeStruct((N, HALF), f32),
            jax.ShapeDtypeStruct((N, HALF), f32),
            jax.ShapeDtypeStruct((N, H), f32),
            jax.ShapeDtypeStruct((N, H), f32),
        ],
    )(agg, den, exp4, b, w, asrc_bd, adst_bd)


def _tc_final_body(agg_ref, den_ref, exp4_ref, b_ref, out_ref):
    x_lo, x_hi = _normalize(agg_ref, den_ref, exp4_ref, b_ref)
    x = jnp.concatenate([x_lo, x_hi], axis=1)
    out_ref[...] = jax.nn.gelu(x, approximate=True)


def _tc_final(agg, den, exp4, b):
    return pl.pallas_call(
        _tc_final_body,
        grid=(N // MBLK,),
        in_specs=[
            pl.BlockSpec((2, MBLK, HALF), lambda i: (0, i, 0)),
            pl.BlockSpec((2, NS, MBLK, HPC), lambda i: (0, 0, i, 0)),
            pl.BlockSpec((HPC, HALF), lambda i: (0, 0)),
            pl.BlockSpec((D,), lambda i: (0,)),
        ],
        out_specs=pl.BlockSpec((MBLK, D), lambda i: (i, 0)),
        out_shape=jax.ShapeDtypeStruct((N, D), jnp.float32),
    )(agg, den, exp4, b)


def _block_diag_att(att):
    rows = jnp.arange(D)
    cols = rows // C
    return jnp.zeros((D, H), jnp.float32).at[rows, cols].set(att.reshape(D))


def kernel(features, edge_indexs, W0, att_src0, att_dst0, b0,
           W1, att_src1, att_dst1, b1):
    f32 = jnp.float32
    i32 = jnp.int32

    loop = jnp.arange(N, dtype=edge_indexs.dtype)
    pad = ET_PAD - ET
    src = jnp.concatenate([edge_indexs[0], loop,
                           jnp.zeros((pad,), edge_indexs.dtype)]).astype(i32)
    dst = jnp.concatenate([edge_indexs[1], loop,
                           jnp.full((pad,), N, edge_indexs.dtype)]).astype(i32)

    exp4 = jnp.repeat(jnp.eye(HPC, dtype=f32), C, axis=1)

    asrc0_bd = _block_diag_att(att_src0)
    adst0_bd = _block_diag_att(att_dst0)
    asrc1_bd = _block_diag_att(att_src1)
    adst1_bd = _block_diag_att(att_dst1)

    def layer(h_lo, h_hi, as_full, ad_full):
        as_lo = as_full[:, :HPC].reshape(-1)
        as_hi = as_full[:, HPC:].reshape(-1)
        ad_lo = ad_full[:, :HPC].reshape(-1)
        ad_hi = ad_full[:, HPC:].reshape(-1)
        w_flat, den_flat = _sc_att(as_lo, as_hi, ad_lo, ad_hi, src, dst)
        agg_flat = _sc_agg(h_lo, h_hi, w_flat, src, dst)
        agg = agg_flat.reshape(2, NP, HALF)
        den = den_flat.reshape(2, NS, NP, HPC)
        return agg, den

    h_lo0, h_hi0, as0, ad0 = _tc_prep(features, W0, asrc0_bd, adst0_bd)
    agg0, den0 = layer(h_lo0, h_hi0, as0, ad0)
    h_lo1, h_hi1, as1, ad1 = _tc_mid(agg0, den0, exp4, b0, W1,
                                     asrc1_bd, adst1_bd)
    agg1, den1 = layer(h_lo1, h_hi1, as1, ad1)
    return _tc_final(agg1, den1, exp4, b1)
